# baseline (speedup 1.0000x reference)
import jax
import jax.numpy as jnp
from jax import lax
from jax.experimental import pallas as pl
from jax.experimental.pallas import tpu as pltpu
from jax.experimental.pallas import tpu_sc as plsc

BATCH = 16384
NFIELDS = 26
VOCAB = 1000000
NDENSE = 13

NC = 2
NS = 16
L = 16
NW = NC * NS
BPW = BATCH // NW
CHUNKS = BPW // L
NIDX = NFIELDS * BPW

TOTAL = NFIELDS * VOCAB
CUT = (TOTAL // 1024) * 1024
NTAIL = TOTAL - (CUT - 1)


def _sc_body(ta_hbm, tb_hbm, idx_hbm, dense_hbm, w_hbm, out_hbm,
             idxa_v, idxb_v, emba_v, embb_v, dense_v, w_v, out_v,
             sema, semb):
    wid = lax.axis_index("s") * NC + lax.axis_index("c")

    pltpu.sync_copy(idx_hbm.at[wid], idxa_v)
    pltpu.sync_copy(dense_hbm.at[wid], dense_v)
    pltpu.sync_copy(w_hbm, w_v)

    for f in range(NFIELDS):
        off = f * VOCAB

        def split(c, _, off=off, f=f):
            sl = pl.ds(f * BPW + c * L, L)
            x = idxa_v[sl]
            if off:
                x = x + off
            idxb_v[sl] = jnp.maximum(x - (CUT - 1), 0)
            idxa_v[sl] = jnp.minimum(x, CUT - 1)
            return 0

        lax.fori_loop(0, CHUNKS, split, 0)

    ga = pltpu.async_copy(ta_hbm.at[idxa_v], emba_v, sema)
    gb = pltpu.async_copy(tb_hbm.at[idxb_v], embb_v, semb)

    wsplats = [w_v[j, :] for j in range(NDENSE)]

    def fc_chunk(c, _):
        sl = pl.ds(c * L, L)
        acc = dense_v[0, sl] * wsplats[0]
        for j in range(1, NDENSE):
            acc = acc + dense_v[j, sl] * wsplats[j]
        out_v[sl] = acc
        return 0

    lax.fori_loop(0, CHUNKS, fc_chunk, 0)

    ga.wait()
    gb.wait()

    def red_chunk(c, _):
        sl = pl.ds(c * L, L)
        acc = out_v[sl]
        for f in range(NFIELDS):
            esl = pl.ds(f * BPW + c * L, L)
            val = jnp.where(idxb_v[esl] > 0, embb_v[esl], emba_v[esl])
            acc = acc + val
        out_v[sl] = acc
        return 0

    lax.fori_loop(0, CHUNKS, red_chunk, 0)

    pltpu.sync_copy(out_v, out_hbm.at[pl.ds(wid * BPW, BPW)])


@jax.jit
def _run(table_a, table_b, idx_rs, dense_rs, w_rep):
    mesh = plsc.VectorSubcoreMesh(core_axis_name="c", subcore_axis_name="s")
    f = pl.kernel(
        _sc_body,
        out_type=jax.ShapeDtypeStruct((BATCH,), jnp.float32),
        mesh=mesh,
        scratch_types=[
            pltpu.VMEM((NIDX,), jnp.int32),
            pltpu.VMEM((NIDX,), jnp.int32),
            pltpu.VMEM((NIDX,), jnp.float32),
            pltpu.VMEM((NIDX,), jnp.float32),
            pltpu.VMEM((NDENSE, BPW), jnp.float32),
            pltpu.VMEM((NDENSE, L), jnp.float32),
            pltpu.VMEM((BPW,), jnp.float32),
            pltpu.SemaphoreType.DMA,
            pltpu.SemaphoreType.DMA,
        ],
    )
    return f(table_a, table_b, idx_rs, dense_rs, w_rep)


def kernel(indices, dense, emb_table, W):
    idx_rs = (indices.T.reshape(NFIELDS, NW, BPW)
              .transpose(1, 0, 2).reshape(NW, NIDX))
    dense_rs = dense.T.reshape(NDENSE, NW, BPW).transpose(1, 0, 2)
    table_a = emb_table[:CUT, :].reshape(-1)
    table_b = emb_table[CUT - 1:, :].reshape(-1)
    w_rep = jnp.broadcast_to(W, (NDENSE, L))
    out = _run(table_a, table_b, idx_rs, dense_rs, w_rep)
    return out.reshape(-1, 1)


# speedup vs baseline: 13.1566x; 13.1566x over previous
import jax
import jax.numpy as jnp
from jax import lax
from jax.experimental import pallas as pl
from jax.experimental.pallas import tpu as pltpu
from jax.experimental.pallas import tpu_sc as plsc

BATCH = 16384
NFIELDS = 26
VOCAB = 1000000
NDENSE = 13

NC = 2
NS = 16
L = 16
NW = NC * NS
BPW = BATCH // NW
CHUNKS = BPW // L

NFA = NFIELDS - 1
NIDXA = NFA * BPW

TOTAL = NFIELDS * VOCAB
CUT = (TOTAL // 1024) * 1024


def _sc_body(ta_hbm, t25_hbm, idx_hbm, idx25_hbm, dense_hbm, w_hbm, out_hbm,
             idxa_v, idx25_v, emba_v, emb25_v, dense_v, w_v, out_v,
             sema, semb):
    wid = lax.axis_index("s") * NC + lax.axis_index("c")

    pltpu.sync_copy(idx_hbm.at[wid], idxa_v)
    pltpu.sync_copy(idx25_hbm.at[wid], idx25_v)
    pltpu.sync_copy(dense_hbm.at[wid], dense_v)
    pltpu.sync_copy(w_hbm, w_v)

    gb = pltpu.async_copy(t25_hbm.at[idx25_v], emb25_v, semb)

    for f in range(1, NFA):
        off = f * VOCAB

        def add_off(c, _, off=off, f=f):
            sl = pl.ds(f * BPW + c * L, L)
            idxa_v[sl] = idxa_v[sl] + off
            return 0

        lax.fori_loop(0, CHUNKS, add_off, 0)

    ga = pltpu.async_copy(ta_hbm.at[idxa_v], emba_v, sema)

    wsplats = [w_v[j, :] for j in range(NDENSE)]

    def fc_chunk(c, _):
        sl = pl.ds(c * L, L)
        acc = dense_v[0, sl] * wsplats[0]
        for j in range(1, NDENSE):
            acc = acc + dense_v[j, sl] * wsplats[j]
        out_v[sl] = acc
        return 0

    lax.fori_loop(0, CHUNKS, fc_chunk, 0)

    ga.wait()
    gb.wait()

    def red_chunk(c, _):
        sl = pl.ds(c * L, L)
        acc = out_v[sl] + emb25_v[sl]
        for f in range(NFA):
            acc = acc + emba_v[pl.ds(f * BPW + c * L, L)]
        out_v[sl] = acc
        return 0

    lax.fori_loop(0, CHUNKS, red_chunk, 0)

    pltpu.sync_copy(out_v, out_hbm.at[pl.ds(wid * BPW, BPW)])


@jax.jit
def _run(table_a, table_25, idx_rs, idx25_rs, dense_rs, w_rep):
    mesh = plsc.VectorSubcoreMesh(core_axis_name="c", subcore_axis_name="s")
    f = pl.kernel(
        _sc_body,
        out_type=jax.ShapeDtypeStruct((BATCH,), jnp.float32),
        mesh=mesh,
        scratch_types=[
            pltpu.VMEM((NIDXA,), jnp.int32),
            pltpu.VMEM((BPW,), jnp.int32),
            pltpu.VMEM((NIDXA,), jnp.float32),
            pltpu.VMEM((BPW,), jnp.float32),
            pltpu.VMEM((NDENSE, BPW), jnp.float32),
            pltpu.VMEM((NDENSE, L), jnp.float32),
            pltpu.VMEM((BPW,), jnp.float32),
            pltpu.SemaphoreType.DMA,
            pltpu.SemaphoreType.DMA,
        ],
    )
    return f(table_a, table_25, idx_rs, idx25_rs, dense_rs, w_rep)


def kernel(indices, dense, emb_table, W):
    i3 = indices.T.reshape(NFIELDS, NW, BPW).transpose(1, 0, 2)
    idx_rs = i3[:, :NFA, :].reshape(NW, NIDXA)
    idx25_rs = i3[:, NFA, :]
    dense_rs = dense.T.reshape(NDENSE, NW, BPW).transpose(1, 0, 2)
    table_a = emb_table[:CUT, :].reshape(-1)
    table_25 = emb_table[NFA * VOCAB:, :].reshape(-1)
    w_rep = jnp.broadcast_to(W, (NDENSE, L))
    out = _run(table_a, table_25, idx_rs, idx25_rs, dense_rs, w_rep)
    return out.reshape(-1, 1)


# speedup vs baseline: 17.0905x; 1.2990x over previous
import jax
import jax.numpy as jnp
from jax import lax
from jax.experimental import pallas as pl
from jax.experimental.pallas import tpu as pltpu
from jax.experimental.pallas import tpu_sc as plsc

BATCH = 16384
NFIELDS = 26
VOCAB = 1000000
NDENSE = 13

NC = 2
NS = 16
L = 16
NW = NC * NS
BPW = BATCH // NW
CHUNKS = BPW // L

NFA = NFIELDS - 1
NIDXA = NFA * BPW

TOTAL = NFIELDS * VOCAB
CUT = (TOTAL // 1024) * 1024
T25_START = TOTAL - 977 * 1024
T25_OFF = NFA * VOCAB - T25_START


def _sc_body(ta_hbm, t25_hbm, idx_hbm, idx25_hbm, dense_hbm, w_hbm, out_hbm,
             idxa_v, idx25_v, emba_v, emb25_v, dense_v, w_v, out_v,
             sema, semb):
    wid = lax.axis_index("s") * NC + lax.axis_index("c")

    pltpu.sync_copy(idx_hbm.at[wid], idxa_v)
    pltpu.sync_copy(idx25_hbm.at[wid], idx25_v)
    pltpu.sync_copy(dense_hbm.at[wid], dense_v)
    pltpu.sync_copy(w_hbm, w_v)

    def add_off25(c, _):
        sl = pl.ds(c * L, L)
        idx25_v[sl] = idx25_v[sl] + T25_OFF
        return 0

    lax.fori_loop(0, CHUNKS, add_off25, 0)
    gb = pltpu.async_copy(t25_hbm.at[idx25_v], emb25_v, semb)

    for f in range(1, NFA):
        off = f * VOCAB

        def add_off(c, _, off=off, f=f):
            sl = pl.ds(f * BPW + c * L, L)
            idxa_v[sl] = idxa_v[sl] + off
            return 0

        lax.fori_loop(0, CHUNKS, add_off, 0)

    ga = pltpu.async_copy(ta_hbm.at[idxa_v], emba_v, sema)

    wsplats = [w_v[j, :] for j in range(NDENSE)]

    def fc_chunk(c, _):
        sl = pl.ds(c * L, L)
        acc = dense_v[0, sl] * wsplats[0]
        for j in range(1, NDENSE):
            acc = acc + dense_v[j, sl] * wsplats[j]
        out_v[sl] = acc
        return 0

    lax.fori_loop(0, CHUNKS, fc_chunk, 0)

    ga.wait()
    gb.wait()

    def red_chunk(c, _):
        sl = pl.ds(c * L, L)
        acc = out_v[sl] + emb25_v[sl]
        for f in range(NFA):
            acc = acc + emba_v[pl.ds(f * BPW + c * L, L)]
        out_v[sl] = acc
        return 0

    lax.fori_loop(0, CHUNKS, red_chunk, 0)

    pltpu.sync_copy(out_v, out_hbm.at[pl.ds(wid * BPW, BPW)])


@jax.jit
def _run(table_a, table_25, idx_rs, idx25_rs, dense_rs, w_rep):
    mesh = plsc.VectorSubcoreMesh(core_axis_name="c", subcore_axis_name="s")
    f = pl.kernel(
        _sc_body,
        out_type=jax.ShapeDtypeStruct((BATCH,), jnp.float32),
        mesh=mesh,
        scratch_types=[
            pltpu.VMEM((NIDXA,), jnp.int32),
            pltpu.VMEM((BPW,), jnp.int32),
            pltpu.VMEM((NIDXA,), jnp.float32),
            pltpu.VMEM((BPW,), jnp.float32),
            pltpu.VMEM((NDENSE, BPW), jnp.float32),
            pltpu.VMEM((NDENSE, L), jnp.float32),
            pltpu.VMEM((BPW,), jnp.float32),
            pltpu.SemaphoreType.DMA,
            pltpu.SemaphoreType.DMA,
        ],
    )
    return f(table_a, table_25, idx_rs, idx25_rs, dense_rs, w_rep)


def kernel(indices, dense, emb_table, W):
    i3 = indices.T.reshape(NFIELDS, NW, BPW).transpose(1, 0, 2)
    idx_rs = i3[:, :NFA, :].reshape(NW, NIDXA)
    idx25_rs = i3[:, NFA, :]
    dense_rs = dense.T.reshape(NDENSE, NW, BPW).transpose(1, 0, 2)
    table_a = emb_table[:CUT, :].reshape(-1)
    table_25 = emb_table[T25_START:, :].reshape(-1)
    w_rep = jnp.broadcast_to(W, (NDENSE, L))
    out = _run(table_a, table_25, idx_rs, idx25_rs, dense_rs, w_rep)
    return out.reshape(-1, 1)


# speedup vs baseline: 17.5735x; 1.0283x over previous
import jax
import jax.numpy as jnp
from jax import lax
from jax.experimental import pallas as pl
from jax.experimental.pallas import tpu as pltpu
from jax.experimental.pallas import tpu_sc as plsc

BATCH = 16384
NFIELDS = 26
VOCAB = 1000000
NDENSE = 13

NC = 2
NS = 16
L = 16
NW = NC * NS
BPW = BATCH // NW
CHUNKS = BPW // L

NFA = NFIELDS - 1
NIDXA = NFA * BPW

CUT = 24415 * 1024
T25_START = NFIELDS * VOCAB - 977 * 1024
T25_OFF = NFA * VOCAB - T25_START


def _sc_body1(t25_hbm, idx25_hbm, dense_hbm, w_hbm, part_hbm,
              idx25_v, emb25_v, dense_v, w_v, out_v, semb):
    wid = lax.axis_index("s") * NC + lax.axis_index("c")

    pltpu.sync_copy(idx25_hbm.at[wid], idx25_v)
    pltpu.sync_copy(dense_hbm.at[wid], dense_v)
    pltpu.sync_copy(w_hbm, w_v)

    def add_off25(c, _):
        sl = pl.ds(c * L, L)
        idx25_v[sl] = idx25_v[sl] + T25_OFF
        return 0

    lax.fori_loop(0, CHUNKS, add_off25, 0)
    gb = pltpu.async_copy(t25_hbm.at[idx25_v], emb25_v, semb)

    wsplats = [w_v[j, :] for j in range(NDENSE)]

    def fc_chunk(c, _):
        sl = pl.ds(c * L, L)
        acc = dense_v[0, sl] * wsplats[0]
        for j in range(1, NDENSE):
            acc = acc + dense_v[j, sl] * wsplats[j]
        out_v[sl] = acc
        return 0

    lax.fori_loop(0, CHUNKS, fc_chunk, 0)
    gb.wait()

    def add25(c, _):
        sl = pl.ds(c * L, L)
        out_v[sl] = out_v[sl] + emb25_v[sl]
        return 0

    lax.fori_loop(0, CHUNKS, add25, 0)
    pltpu.sync_copy(out_v, part_hbm.at[pl.ds(wid * BPW, BPW)])


def _sc_body2(ta_hbm, idx_hbm, part_hbm, out_hbm,
              idxa_v, emba_v, out_v, sema):
    wid = lax.axis_index("s") * NC + lax.axis_index("c")

    pltpu.sync_copy(idx_hbm.at[wid], idxa_v)

    for f in range(1, NFA):
        off = f * VOCAB

        def add_off(c, _, off=off, f=f):
            sl = pl.ds(f * BPW + c * L, L)
            idxa_v[sl] = idxa_v[sl] + off
            return 0

        lax.fori_loop(0, CHUNKS, add_off, 0)

    ga = pltpu.async_copy(ta_hbm.at[idxa_v], emba_v, sema)

    pltpu.sync_copy(part_hbm.at[pl.ds(wid * BPW, BPW)], out_v)
    ga.wait()

    def red_chunk(c, _):
        sl = pl.ds(c * L, L)
        acc = out_v[sl]
        for f in range(NFA):
            acc = acc + emba_v[pl.ds(f * BPW + c * L, L)]
        out_v[sl] = acc
        return 0

    lax.fori_loop(0, CHUNKS, red_chunk, 0)
    pltpu.sync_copy(out_v, out_hbm.at[pl.ds(wid * BPW, BPW)])


@jax.jit
def _run(table_a, table_25, idx_rs, idx25_rs, dense_rs, w_rep):
    mesh = plsc.VectorSubcoreMesh(core_axis_name="c", subcore_axis_name="s")
    k1 = pl.kernel(
        _sc_body1,
        out_type=jax.ShapeDtypeStruct((BATCH,), jnp.float32),
        mesh=mesh,
        scratch_types=[
            pltpu.VMEM((BPW,), jnp.int32),
            pltpu.VMEM((BPW,), jnp.float32),
            pltpu.VMEM((NDENSE, BPW), jnp.float32),
            pltpu.VMEM((NDENSE, L), jnp.float32),
            pltpu.VMEM((BPW,), jnp.float32),
            pltpu.SemaphoreType.DMA,
        ],
    )
    k2 = pl.kernel(
        _sc_body2,
        out_type=jax.ShapeDtypeStruct((BATCH,), jnp.float32),
        mesh=mesh,
        scratch_types=[
            pltpu.VMEM((NIDXA,), jnp.int32),
            pltpu.VMEM((NIDXA,), jnp.float32),
            pltpu.VMEM((BPW,), jnp.float32),
            pltpu.SemaphoreType.DMA,
        ],
    )
    partial = k1(table_25, idx25_rs, dense_rs, w_rep)
    return k2(table_a, idx_rs, partial)


def kernel(indices, dense, emb_table, W):
    i3 = indices.T.reshape(NFIELDS, NW, BPW).transpose(1, 0, 2)
    idx_rs = i3[:, :NFA, :].reshape(NW, NIDXA)
    idx25_rs = i3[:, NFA, :]
    dense_rs = dense.T.reshape(NDENSE, NW, BPW).transpose(1, 0, 2)
    table_a = emb_table[:CUT, :].reshape(-1)
    table_25 = emb_table[T25_START:, :].reshape(-1)
    w_rep = jnp.broadcast_to(W, (NDENSE, L))
    out = _run(table_a, table_25, idx_rs, idx25_rs, dense_rs, w_rep)
    return out.reshape(-1, 1)


# speedup vs baseline: 18.5234x; 1.0541x over previous
import jax
import jax.numpy as jnp
from jax import lax
from jax.experimental import pallas as pl
from jax.experimental.pallas import tpu as pltpu
from jax.experimental.pallas import tpu_sc as plsc

BATCH = 16384
NFIELDS = 26
VOCAB = 1000000
NDENSE = 13

NC = 2
NS = 16
L = 16
NW = NC * NS
BPW = BATCH // NW
CHUNKS = BPW // L

FLO = 12
FHI = NFIELDS - FLO
NLO = FLO * BPW
NHI = FHI * BPW

A1_LEN = 11719 * 1024
A2_START = 11_999_872
A2_LEN = NFIELDS * VOCAB - A2_START


def _sc_body_hi(ta_hbm, idx_hbm, dense_hbm, w_hbm, part_hbm,
                idx_v, emb_v, dense_v, w_v, out_v, sem):
    wid = lax.axis_index("s") * NC + lax.axis_index("c")

    pltpu.sync_copy(idx_hbm.at[wid], idx_v)
    pltpu.sync_copy(dense_hbm.at[wid], dense_v)
    pltpu.sync_copy(w_hbm, w_v)

    for f in range(FHI):
        off = (FLO + f) * VOCAB - A2_START

        def add_off(c, _, off=off, f=f):
            sl = pl.ds(f * BPW + c * L, L)
            idx_v[sl] = idx_v[sl] + off
            return 0

        lax.fori_loop(0, CHUNKS, add_off, 0)

    g = pltpu.async_copy(ta_hbm.at[idx_v], emb_v, sem)

    wsplats = [w_v[j, :] for j in range(NDENSE)]

    def fc_chunk(c, _):
        sl = pl.ds(c * L, L)
        acc = dense_v[0, sl] * wsplats[0]
        for j in range(1, NDENSE):
            acc = acc + dense_v[j, sl] * wsplats[j]
        out_v[sl] = acc
        return 0

    lax.fori_loop(0, CHUNKS, fc_chunk, 0)
    g.wait()

    def red_chunk(c, _):
        sl = pl.ds(c * L, L)
        acc = out_v[sl]
        for f in range(FHI):
            acc = acc + emb_v[pl.ds(f * BPW + c * L, L)]
        out_v[sl] = acc
        return 0

    lax.fori_loop(0, CHUNKS, red_chunk, 0)
    pltpu.sync_copy(out_v, part_hbm.at[pl.ds(wid * BPW, BPW)])


def _sc_body_lo(ta_hbm, idx_hbm, part_hbm, out_hbm,
                idx_v, emb_v, out_v, sem):
    wid = lax.axis_index("s") * NC + lax.axis_index("c")

    pltpu.sync_copy(idx_hbm.at[wid], idx_v)

    for f in range(1, FLO):
        off = f * VOCAB

        def add_off(c, _, off=off, f=f):
            sl = pl.ds(f * BPW + c * L, L)
            idx_v[sl] = idx_v[sl] + off
            return 0

        lax.fori_loop(0, CHUNKS, add_off, 0)

    g = pltpu.async_copy(ta_hbm.at[idx_v], emb_v, sem)

    pltpu.sync_copy(part_hbm.at[pl.ds(wid * BPW, BPW)], out_v)
    g.wait()

    def red_chunk(c, _):
        sl = pl.ds(c * L, L)
        acc = out_v[sl]
        for f in range(FLO):
            acc = acc + emb_v[pl.ds(f * BPW + c * L, L)]
        out_v[sl] = acc
        return 0

    lax.fori_loop(0, CHUNKS, red_chunk, 0)
    pltpu.sync_copy(out_v, out_hbm.at[pl.ds(wid * BPW, BPW)])


@jax.jit
def _run(table_a1, table_a2, idx_lo, idx_hi, dense_rs, w_rep):
    mesh = plsc.VectorSubcoreMesh(core_axis_name="c", subcore_axis_name="s")
    k_hi = pl.kernel(
        _sc_body_hi,
        out_type=jax.ShapeDtypeStruct((BATCH,), jnp.float32),
        mesh=mesh,
        scratch_types=[
            pltpu.VMEM((NHI,), jnp.int32),
            pltpu.VMEM((NHI,), jnp.float32),
            pltpu.VMEM((NDENSE, BPW), jnp.float32),
            pltpu.VMEM((NDENSE, L), jnp.float32),
            pltpu.VMEM((BPW,), jnp.float32),
            pltpu.SemaphoreType.DMA,
        ],
    )
    k_lo = pl.kernel(
        _sc_body_lo,
        out_type=jax.ShapeDtypeStruct((BATCH,), jnp.float32),
        mesh=mesh,
        scratch_types=[
            pltpu.VMEM((NLO,), jnp.int32),
            pltpu.VMEM((NLO,), jnp.float32),
            pltpu.VMEM((BPW,), jnp.float32),
            pltpu.SemaphoreType.DMA,
        ],
    )
    partial = k_hi(table_a2, idx_hi, dense_rs, w_rep)
    return k_lo(table_a1, idx_lo, partial)


def kernel(indices, dense, emb_table, W):
    i3 = indices.T.reshape(NFIELDS, NW, BPW).transpose(1, 0, 2)
    idx_lo = i3[:, :FLO, :].reshape(NW, NLO)
    idx_hi = i3[:, FLO:, :].reshape(NW, NHI)
    dense_rs = dense.T.reshape(NDENSE, NW, BPW).transpose(1, 0, 2)
    table_a1 = emb_table[:A1_LEN, :].reshape(-1)
    table_a2 = emb_table[A2_START:, :].reshape(-1)
    w_rep = jnp.broadcast_to(W, (NDENSE, L))
    out = _run(table_a1, table_a2, idx_lo, idx_hi, dense_rs, w_rep)
    return out.reshape(-1, 1)


# speedup vs baseline: 18.5350x; 1.0006x over previous
import jax
import jax.numpy as jnp
from jax import lax
from jax.experimental import pallas as pl
from jax.experimental.pallas import tpu as pltpu
from jax.experimental.pallas import tpu_sc as plsc

BATCH = 16384
NFIELDS = 26
VOCAB = 1000000
NDENSE = 13

NC = 2
NS = 16
L = 16
NW = NC * NS
BPW = BATCH // NW
CHUNKS = BPW // L

FLO = 8
FHI = NFIELDS - FLO
NLO = FLO * BPW
NHI = FHI * BPW

A1_LEN = 7813 * 1024
A2_START = 7_999_104
A2_LEN = NFIELDS * VOCAB - A2_START


def _sc_body_hi(ta_hbm, idx_hbm, dense_hbm, w_hbm, part_hbm,
                idx_v, emb_v, dense_v, w_v, out_v, sem):
    wid = lax.axis_index("s") * NC + lax.axis_index("c")

    pltpu.sync_copy(idx_hbm.at[wid], idx_v)
    pltpu.sync_copy(dense_hbm.at[wid], dense_v)
    pltpu.sync_copy(w_hbm, w_v)

    def add_off_hi(c, _):
        for f in range(FHI):
            off = (FLO + f) * VOCAB - A2_START
            sl = pl.ds(f * BPW + c * L, L)
            idx_v[sl] = idx_v[sl] + off
        return 0

    lax.fori_loop(0, CHUNKS, add_off_hi, 0)

    g = pltpu.async_copy(ta_hbm.at[idx_v], emb_v, sem)

    wsplats = [w_v[j, :] for j in range(NDENSE)]

    def fc_chunk(c, _):
        sl = pl.ds(c * L, L)
        acc = dense_v[0, sl] * wsplats[0]
        for j in range(1, NDENSE):
            acc = acc + dense_v[j, sl] * wsplats[j]
        out_v[sl] = acc
        return 0

    lax.fori_loop(0, CHUNKS, fc_chunk, 0)
    g.wait()

    def red_chunk(c, _):
        sl = pl.ds(c * L, L)
        acc = out_v[sl]
        for f in range(FHI):
            acc = acc + emb_v[pl.ds(f * BPW + c * L, L)]
        out_v[sl] = acc
        return 0

    lax.fori_loop(0, CHUNKS, red_chunk, 0)
    pltpu.sync_copy(out_v, part_hbm.at[pl.ds(wid * BPW, BPW)])


def _sc_body_lo(ta_hbm, idx_hbm, part_hbm, out_hbm,
                idx_v, emb_v, out_v, sem):
    wid = lax.axis_index("s") * NC + lax.axis_index("c")

    pltpu.sync_copy(idx_hbm.at[wid], idx_v)

    def add_off_lo(c, _):
        for f in range(1, FLO):
            off = f * VOCAB
            sl = pl.ds(f * BPW + c * L, L)
            idx_v[sl] = idx_v[sl] + off
        return 0

    lax.fori_loop(0, CHUNKS, add_off_lo, 0)

    g = pltpu.async_copy(ta_hbm.at[idx_v], emb_v, sem)

    pltpu.sync_copy(part_hbm.at[pl.ds(wid * BPW, BPW)], out_v)
    g.wait()

    def red_chunk(c, _):
        sl = pl.ds(c * L, L)
        acc = out_v[sl]
        for f in range(FLO):
            acc = acc + emb_v[pl.ds(f * BPW + c * L, L)]
        out_v[sl] = acc
        return 0

    lax.fori_loop(0, CHUNKS, red_chunk, 0)
    pltpu.sync_copy(out_v, out_hbm.at[pl.ds(wid * BPW, BPW)])


@jax.jit
def _run(table_a1, table_a2, idx_lo, idx_hi, dense_rs, w_rep):
    mesh = plsc.VectorSubcoreMesh(core_axis_name="c", subcore_axis_name="s")
    k_hi = pl.kernel(
        _sc_body_hi,
        out_type=jax.ShapeDtypeStruct((BATCH,), jnp.float32),
        mesh=mesh,
        scratch_types=[
            pltpu.VMEM((NHI,), jnp.int32),
            pltpu.VMEM((NHI,), jnp.float32),
            pltpu.VMEM((NDENSE, BPW), jnp.float32),
            pltpu.VMEM((NDENSE, L), jnp.float32),
            pltpu.VMEM((BPW,), jnp.float32),
            pltpu.SemaphoreType.DMA,
        ],
    )
    k_lo = pl.kernel(
        _sc_body_lo,
        out_type=jax.ShapeDtypeStruct((BATCH,), jnp.float32),
        mesh=mesh,
        scratch_types=[
            pltpu.VMEM((NLO,), jnp.int32),
            pltpu.VMEM((NLO,), jnp.float32),
            pltpu.VMEM((BPW,), jnp.float32),
            pltpu.SemaphoreType.DMA,
        ],
    )
    partial = k_hi(table_a2, idx_hi, dense_rs, w_rep)
    return k_lo(table_a1, idx_lo, partial)


def kernel(indices, dense, emb_table, W):
    i3 = indices.T.reshape(NFIELDS, NW, BPW).transpose(1, 0, 2)
    idx_lo = i3[:, :FLO, :].reshape(NW, NLO)
    idx_hi = i3[:, FLO:, :].reshape(NW, NHI)
    dense_rs = dense.T.reshape(NDENSE, NW, BPW).transpose(1, 0, 2)
    table_a1 = emb_table[:A1_LEN, :].reshape(-1)
    table_a2 = emb_table[A2_START:, :].reshape(-1)
    w_rep = jnp.broadcast_to(W, (NDENSE, L))
    out = _run(table_a1, table_a2, idx_lo, idx_hi, dense_rs, w_rep)
    return out.reshape(-1, 1)


# speedup vs baseline: 18.9857x; 1.0243x over previous
import jax
import jax.numpy as jnp
from jax import lax
from jax.experimental import pallas as pl
from jax.experimental.pallas import tpu as pltpu
from jax.experimental.pallas import tpu_sc as plsc

BATCH = 16384
NFIELDS = 26
VOCAB = 1000000
NDENSE = 13

NC = 2
NS = 16
L = 16
NW = NC * NS
BPW = BATCH // NW
CHUNKS = BPW // L

FLO = 10
FHI = NFIELDS - FLO
NLO = FLO * BPW
NHI = FHI * BPW

A1_LEN = 9766 * 1024
A2_START = 10_000_000
A2_LEN = NFIELDS * VOCAB - A2_START


def _sc_body_hi(ta_hbm, idx_hbm, dense_hbm, w_hbm, part_hbm,
                idx_v, emb_v, dense_v, w_v, out_v, sem):
    wid = lax.axis_index("s") * NC + lax.axis_index("c")

    pltpu.sync_copy(idx_hbm.at[wid], idx_v)
    pltpu.sync_copy(dense_hbm.at[wid], dense_v)
    pltpu.sync_copy(w_hbm, w_v)

    def add_off_hi(c, _):
        for f in range(FHI):
            off = (FLO + f) * VOCAB - A2_START
            sl = pl.ds(f * BPW + c * L, L)
            idx_v[sl] = idx_v[sl] + off
        return 0

    lax.fori_loop(0, CHUNKS, add_off_hi, 0)

    g = pltpu.async_copy(ta_hbm.at[idx_v], emb_v, sem)

    wsplats = [w_v[j, :] for j in range(NDENSE)]

    def fc_chunk(c, _):
        sl = pl.ds(c * L, L)
        acc = dense_v[0, sl] * wsplats[0]
        for j in range(1, NDENSE):
            acc = acc + dense_v[j, sl] * wsplats[j]
        out_v[sl] = acc
        return 0

    lax.fori_loop(0, CHUNKS, fc_chunk, 0)
    g.wait()

    def red_chunk(c, _):
        sl = pl.ds(c * L, L)
        acc = out_v[sl]
        for f in range(FHI):
            acc = acc + emb_v[pl.ds(f * BPW + c * L, L)]
        out_v[sl] = acc
        return 0

    lax.fori_loop(0, CHUNKS, red_chunk, 0)
    pltpu.sync_copy(out_v, part_hbm.at[pl.ds(wid * BPW, BPW)])


def _sc_body_lo(ta_hbm, idx_hbm, part_hbm, out_hbm,
                idx_v, emb_v, out_v, sem):
    wid = lax.axis_index("s") * NC + lax.axis_index("c")

    pltpu.sync_copy(idx_hbm.at[wid], idx_v)

    def add_off_lo(c, _):
        for f in range(1, FLO):
            off = f * VOCAB
            sl = pl.ds(f * BPW + c * L, L)
            idx_v[sl] = idx_v[sl] + off
        return 0

    lax.fori_loop(0, CHUNKS, add_off_lo, 0)

    g = pltpu.async_copy(ta_hbm.at[idx_v], emb_v, sem)

    pltpu.sync_copy(part_hbm.at[pl.ds(wid * BPW, BPW)], out_v)
    g.wait()

    def red_chunk(c, _):
        sl = pl.ds(c * L, L)
        acc = out_v[sl]
        for f in range(FLO):
            acc = acc + emb_v[pl.ds(f * BPW + c * L, L)]
        out_v[sl] = acc
        return 0

    lax.fori_loop(0, CHUNKS, red_chunk, 0)
    pltpu.sync_copy(out_v, out_hbm.at[pl.ds(wid * BPW, BPW)])


@jax.jit
def _run(table_a1, table_a2, idx_lo, idx_hi, dense_rs, w_rep):
    mesh = plsc.VectorSubcoreMesh(core_axis_name="c", subcore_axis_name="s")
    k_hi = pl.kernel(
        _sc_body_hi,
        out_type=jax.ShapeDtypeStruct((BATCH,), jnp.float32),
        mesh=mesh,
        scratch_types=[
            pltpu.VMEM((NHI,), jnp.int32),
            pltpu.VMEM((NHI,), jnp.float32),
            pltpu.VMEM((NDENSE, BPW), jnp.float32),
            pltpu.VMEM((NDENSE, L), jnp.float32),
            pltpu.VMEM((BPW,), jnp.float32),
            pltpu.SemaphoreType.DMA,
        ],
    )
    k_lo = pl.kernel(
        _sc_body_lo,
        out_type=jax.ShapeDtypeStruct((BATCH,), jnp.float32),
        mesh=mesh,
        scratch_types=[
            pltpu.VMEM((NLO,), jnp.int32),
            pltpu.VMEM((NLO,), jnp.float32),
            pltpu.VMEM((BPW,), jnp.float32),
            pltpu.SemaphoreType.DMA,
        ],
    )
    partial = k_hi(table_a2, idx_hi, dense_rs, w_rep)
    return k_lo(table_a1, idx_lo, partial)


def kernel(indices, dense, emb_table, W):
    i3 = indices.T.reshape(NFIELDS, NW, BPW).transpose(1, 0, 2)
    idx_lo = i3[:, :FLO, :].reshape(NW, NLO)
    idx_hi = i3[:, FLO:, :].reshape(NW, NHI)
    dense_rs = dense.T.reshape(NDENSE, NW, BPW).transpose(1, 0, 2)
    table_a1 = emb_table[:A1_LEN, :].reshape(-1)
    table_a2 = emb_table[A2_START:, :].reshape(-1)
    w_rep = jnp.broadcast_to(W, (NDENSE, L))
    out = _run(table_a1, table_a2, idx_lo, idx_hi, dense_rs, w_rep)
    return out.reshape(-1, 1)
